# baseline (speedup 1.0000x reference)
import functools

import jax
import jax.numpy as jnp
from jax import lax
from jax.experimental import pallas as pl
from jax.experimental.pallas import tpu as pltpu
from jax.experimental.pallas import tpu_sc as plsc


def kernel(discrete_actions, table):
    B, T, A = discrete_actions.shape
    _, MB, D = table.shape

    info = plsc.get_sparse_core_info()
    NC, NS, L = info.num_cores, info.num_subcores, info.num_lanes
    NW = NC * NS
    BC = B // NW

    idxT = discrete_actions.transpose(2, 0, 1)
    idxT = idxT.reshape(A, NW, BC, T).transpose(0, 1, 3, 2)
    idx_lin = idxT.reshape(A * NW * T * BC).astype(jnp.int32)

    tab_lin = jnp.swapaxes(table, 1, 2).reshape(A * D * MB)

    mesh = plsc.VectorSubcoreMesh(core_axis_name="c", subcore_axis_name="s")

    @functools.partial(
        pl.kernel,
        out_type=jax.ShapeDtypeStruct((T, A, D, B), jnp.float32),
        mesh=mesh,
        compiler_params=pltpu.CompilerParams(needs_layout_passes=False),
        scratch_types=[
            pltpu.VMEM((D * MB,), jnp.float32),
            pltpu.VMEM((T * BC,), jnp.int32),
            pltpu.VMEM((4, D, BC), jnp.float32),
            pltpu.SemaphoreType.DMA,
        ],
    )
    def _gather(idx_hbm, tab_hbm, out_hbm, tab_v, idxc_v, outbuf_v, osem):
        wid = lax.axis_index("s") * NC + lax.axis_index("c")

        def wait_one_block():
            pltpu.make_async_copy(
                outbuf_v.at[0],
                out_hbm.at[0, 0, :, pl.ds(wid * BC, BC)],
                osem,
            ).wait()

        def half(a, t, p):
            g = a * T + t
            @pl.when(g >= 4)
            def _():
                wait_one_block()
            NB = BC // L
            bins = [idxc_v[pl.ds(t * BC + c * L, L)] for c in range(NB)]

            def load_row(d):
                ref = tab_v.at[pl.ds(d * MB, MB)]
                return [plsc.load_gather(ref, [b]) for b in bins]

            prev = load_row(0)
            for d in range(1, D):
                ref = tab_v.at[pl.ds(d * MB, MB)]
                cur = []
                for c in range(NB):
                    cur.append(plsc.load_gather(ref, [bins[c]]))
                    outbuf_v[p, d - 1, pl.ds(c * L, L)] = prev[c]
                prev = cur
            for c in range(NB):
                outbuf_v[p, D - 1, pl.ds(c * L, L)] = prev[c]
            pltpu.async_copy(
                outbuf_v.at[p],
                out_hbm.at[t, a, :, pl.ds(wid * BC, BC)],
                osem,
            )

        def one_axis(a, off):
            pltpu.sync_copy(tab_hbm.at[pl.ds(a * D * MB, D * MB)], tab_v)
            pltpu.sync_copy(
                idx_hbm.at[pl.ds((a * NW + wid) * T * BC, T * BC)], idxc_v)

            def t_body(tq, _):
                for i in range(4):
                    half(a, 4 * tq + i, (i + off) % 4)
                return 0

            lax.fori_loop(0, T // 4, t_body, 0)
            half(a, T - 2, off % 4)
            half(a, T - 1, (1 + off) % 4)

        def a_body(q, _):
            one_axis(2 * q, 0)
            one_axis(2 * q + 1, 2)
            return 0

        lax.fori_loop(0, A // 2, a_body, 0)
        for _ in range(4):
            wait_one_block()

    out_t = _gather(idx_lin, tab_lin)
    return jnp.transpose(out_t, (3, 0, 1, 2))


# speedup vs baseline: 1.6370x; 1.6370x over previous
import functools

import jax
import jax.numpy as jnp
from jax import lax
from jax.experimental import pallas as pl
from jax.experimental.pallas import tpu as pltpu
from jax.experimental.pallas import tpu_sc as plsc


def kernel(discrete_actions, table):
    B, T, A = discrete_actions.shape
    _, MB, D = table.shape

    info = plsc.get_sparse_core_info()
    NC, NS, L = info.num_cores, info.num_subcores, info.num_lanes
    NW = NC * NS
    BC = B // NW

    idxT = discrete_actions.transpose(2, 0, 1)
    idxT = idxT.reshape(A, NW, BC, T).transpose(0, 1, 3, 2)
    idx_lin = idxT.reshape(A * NW * T * BC).astype(jnp.int32)

    tab_lin = jnp.swapaxes(table, 1, 2).reshape(A * D * MB)

    mesh = plsc.VectorSubcoreMesh(core_axis_name="c", subcore_axis_name="s")

    @functools.partial(
        pl.kernel,
        out_type=jax.ShapeDtypeStruct((T, A, D, B), jnp.float32),
        mesh=mesh,
        compiler_params=pltpu.CompilerParams(needs_layout_passes=False),
        scratch_types=[
            pltpu.VMEM((D * MB,), jnp.float32),
            pltpu.VMEM((T * BC,), jnp.int32),
            pltpu.VMEM((2, D, BC), jnp.float32),
            pltpu.SemaphoreType.DMA,
        ],
    )
    def _gather(idx_hbm, tab_hbm, out_hbm, tab_v, idxc_v, outbuf_v, osem):
        wid = lax.axis_index("s") * NC + lax.axis_index("c")

        def wait_one_block():
            pltpu.make_async_copy(
                outbuf_v.at[0],
                out_hbm.at[0, 0, :, pl.ds(wid * BC, BC)],
                osem,
            ).wait()

        def half(a, t, p):
            g = a * T + t
            @pl.when(g >= 2)
            def _():
                wait_one_block()
            NB = BC // L
            bins = [idxc_v[pl.ds(t * BC + c * L, L)] for c in range(NB)]

            def load_row(d):
                ref = tab_v.at[pl.ds(d * MB, MB)]
                return [plsc.load_gather(ref, [b]) for b in bins]

            prev = load_row(0)
            for d in range(1, D):
                ref = tab_v.at[pl.ds(d * MB, MB)]
                cur = []
                for c in range(NB):
                    cur.append(plsc.load_gather(ref, [bins[c]]))
                    outbuf_v[p, d - 1, pl.ds(c * L, L)] = prev[c]
                prev = cur
            for c in range(NB):
                outbuf_v[p, D - 1, pl.ds(c * L, L)] = prev[c]
            pltpu.async_copy(
                outbuf_v.at[p],
                out_hbm.at[t, a, :, pl.ds(wid * BC, BC)],
                osem,
            )

        def a_body(a, _):
            pltpu.sync_copy(tab_hbm.at[pl.ds(a * D * MB, D * MB)], tab_v)
            pltpu.sync_copy(
                idx_hbm.at[pl.ds((a * NW + wid) * T * BC, T * BC)], idxc_v)

            def t_body(tp, _):
                half(a, 2 * tp, 0)
                half(a, 2 * tp + 1, 1)
                return 0

            lax.fori_loop(0, T // 2, t_body, 0)
            return 0

        lax.fori_loop(0, A, a_body, 0)
        wait_one_block()
        wait_one_block()

    out_t = _gather(idx_lin, tab_lin)
    return jnp.transpose(out_t, (3, 0, 1, 2))
